# Initial kernel scaffold; baseline (speedup 1.0000x reference)
import jax
import jax.numpy as jnp
from jax.experimental import pallas as pl


def kernel(x, edge_index, W, b):
    raise NotImplementedError("write your pallas kernel here")


# speedup vs baseline: 1.0652x; 1.0652x over previous
import jax
import jax.numpy as jnp
from jax import lax
from jax.experimental import pallas as pl
from jax.experimental.pallas import tpu as pltpu
from jax.experimental.pallas import tpu_sc as plsc

N = 10000
NP = 10240
E = 320000
D = 128
NC = 2
NS = 16
NT = NC * NS
EPT = E // NT
CH = 80
NCHUNK = EPT // CH
MB = CH // 16
RPT = NP // NS
CR = NP // D
DUNROLL = 8


def _sqrt16(q):
    bits = plsc.bitcast(q, jnp.int32)
    y = plsc.bitcast((bits >> 1) + jnp.int32(0x1FBD1DF5), jnp.float32)
    y = 0.5 * (y + q / y)
    y = 0.5 * (y + q / y)
    y = 0.5 * (y + q / y)
    return y


def _sc_body(xneg_hbm, xpos_hbm, row_hbm, col_hbm, nrm_hbm,
             parts_hbm, cnts_hbm,
             ri, ci, xi, xj, msg, niv, njv, cnt2d, cidx, shared, shared_cnt,
             sem1, sem2, sem3, sem4):
    cid = lax.axis_index("c")
    sid = lax.axis_index("s")
    wid = cid * NS + sid

    zero16 = jnp.zeros((16,), jnp.float32)

    def zrow(e, c):
        for kk in range(D // 16):
            msg[e, pl.ds(kk * 16, 16)] = zero16
            cnt2d[e, pl.ds(kk * 16, 16)] = zero16
        return c

    lax.fori_loop(0, CH, zrow, 0)

    iota16 = lax.iota(jnp.int32, 16)

    def irow(k, c):
        cidx[pl.ds(k * 16, 16)] = iota16 + k * 16
        return c

    lax.fori_loop(0, CR // 16, irow, 0)

    base_r = sid * RPT

    def zcp(t, c):
        pltpu.sync_copy(msg, shared.at[pl.ds(base_r + t * CH, CH)])
        return c

    lax.fori_loop(0, RPT // CH, zcp, 0)

    @pl.when(sid == 0)
    def _():
        pltpu.sync_copy(msg, shared_cnt)

    plsc.subcore_barrier()

    ebase0 = wid * EPT

    def chunk(g, c):
        base = ebase0 + g * CH
        pltpu.sync_copy(row_hbm.at[pl.ds(base, CH)], ri)
        pltpu.sync_copy(col_hbm.at[pl.ds(base, CH)], ci)
        c1 = pltpu.async_copy(xneg_hbm.at[ri], xi, sem1)
        c2 = pltpu.async_copy(xpos_hbm.at[ci], xj, sem2)
        c3 = pltpu.async_copy(nrm_hbm.at[ri], niv, sem3)
        c4 = pltpu.async_copy(nrm_hbm.at[ci], njv, sem4)
        c1.wait()
        c2.wait()
        c3.wait()
        c4.wait()
        for m in range(MB):
            r16 = ri[pl.ds(m * 16, 16)]
            ni = niv[pl.ds(m * 16, 16)]
            nj = njv[pl.ds(m * 16, 16)]
            ev = iota16 + m * 16

            def dbody(k, acc):
                for u in range(DUNROLL):
                    dv = jnp.full((16,), k * DUNROLL + u, jnp.int32)
                    a = plsc.load_gather(xi, [ev, dv])
                    b_ = plsc.load_gather(xj, [ev, dv])
                    acc = acc + a * b_
                return acc

            dot = lax.fori_loop(0, D // DUNROLL, dbody, zero16)
            quad = 1.0 - (dot * dot) / (ni * nj + 1e-9)
            dist = _sqrt16(jnp.maximum(jnp.abs(quad), 1e-9))
            w = jnp.exp(-dist)

            def mbody(k, cc):
                for u in range(DUNROLL):
                    dv = jnp.full((16,), k * DUNROLL + u, jnp.int32)
                    b_ = plsc.load_gather(xj, [ev, dv])
                    plsc.store_scatter(msg, [ev, dv], b_ * w)
                return cc

            lax.fori_loop(0, D // DUNROLL, mbody, 0)

            cnt, last = plsc.scan_count(r16)
            plsc.addupdate_scatter(cnt2d, [r16 >> 7, r16 & 127],
                                   cnt.astype(jnp.float32), mask=last)
        pltpu.sync_copy(msg, shared.at[ri], add=True)
        return c

    lax.fori_loop(0, NCHUNK, chunk, 0)

    pltpu.sync_copy(cnt2d, shared_cnt.at[cidx], add=True)

    plsc.subcore_barrier()
    pltpu.sync_copy(shared.at[pl.ds(base_r, RPT)],
                    parts_hbm.at[cid, pl.ds(base_r, RPT)])

    @pl.when(sid == 0)
    def _():
        pltpu.sync_copy(shared_cnt, cnts_hbm.at[cid])


_sc_agg = pl.kernel(
    _sc_body,
    out_type=[
        jax.ShapeDtypeStruct((NC, NP, D), jnp.float32),
        jax.ShapeDtypeStruct((NC, CR, D), jnp.float32),
    ],
    mesh=plsc.VectorSubcoreMesh(core_axis_name="c", subcore_axis_name="s"),
    compiler_params=pltpu.CompilerParams(needs_layout_passes=False),
    scratch_types=[
        pltpu.VMEM((CH,), jnp.int32),
        pltpu.VMEM((CH,), jnp.int32),
        pltpu.VMEM((CH, D), jnp.float32),
        pltpu.VMEM((CH, D), jnp.float32),
        pltpu.VMEM((CH, D), jnp.float32),
        pltpu.VMEM((CH,), jnp.float32),
        pltpu.VMEM((CH,), jnp.float32),
        pltpu.VMEM((CR, D), jnp.float32),
        pltpu.VMEM((CR,), jnp.int32),
        pltpu.VMEM_SHARED((NP, D), jnp.float32),
        pltpu.VMEM_SHARED((CR, D), jnp.float32),
        pltpu.SemaphoreType.DMA,
        pltpu.SemaphoreType.DMA,
        pltpu.SemaphoreType.DMA,
        pltpu.SemaphoreType.DMA,
    ],
)


def _prep_body(x_ref, xneg_ref, nrm_ref):
    xv = x_ref[...]
    sq = jnp.sum(xv * xv, axis=1)
    nrm_ref[...] = sq - 2.0 * (xv[:, D - 1] * xv[:, D - 1])
    sign = jnp.where(lax.broadcasted_iota(jnp.int32, (1, D), 1) == D - 1,
                     -1.0, 1.0).astype(jnp.float32)
    xneg_ref[...] = xv * sign


def _prep(x):
    return pl.pallas_call(
        _prep_body,
        out_shape=[
            jax.ShapeDtypeStruct((N, D), jnp.float32),
            jax.ShapeDtypeStruct((N,), jnp.float32),
        ],
    )(x)


BR = 2048


def _post_body(p0_ref, p1_ref, c0_ref, c1_ref, w_ref, b_ref, o_ref):
    agg = p0_ref[...] + p1_ref[...]
    cnt = jnp.maximum(c0_ref[...] + c1_ref[...], 1.0)
    mean = agg / cnt
    y = lax.dot_general(mean, w_ref[...], (((1,), (1,)), ((), ())),
                        preferred_element_type=jnp.float32)
    y = y + b_ref[...]
    nr = jnp.sqrt(jnp.sum(y * y, axis=1, keepdims=True))
    o_ref[...] = y / (nr + 1e-8)


def _post(p0, p1, c0, c1, W, b2):
    return pl.pallas_call(
        _post_body,
        grid=(NP // BR,),
        in_specs=[
            pl.BlockSpec((BR, D), lambda i: (i, 0)),
            pl.BlockSpec((BR, D), lambda i: (i, 0)),
            pl.BlockSpec((BR, 1), lambda i: (i, 0)),
            pl.BlockSpec((BR, 1), lambda i: (i, 0)),
            pl.BlockSpec((D, D), lambda i: (0, 0)),
            pl.BlockSpec((1, D), lambda i: (0, 0)),
        ],
        out_specs=pl.BlockSpec((BR, D), lambda i: (i, 0)),
        out_shape=jax.ShapeDtypeStruct((NP, D), jnp.float32),
    )(p0, p1, c0, c1, W, b2)


def kernel(x, edge_index, W, b):
    row = edge_index[0]
    col = edge_index[1]
    xneg, nrm = _prep(x)
    parts, cnts = _sc_agg(xneg, x, row, col, nrm)
    c0 = cnts[0].reshape(NP, 1)
    c1 = cnts[1].reshape(NP, 1)
    return _post(parts[0], parts[1], c0, c1, W, b.reshape(1, D))[:N]
